# baseline (speedup 1.0000x reference)
import functools

import jax
import jax.numpy as jnp
from jax import lax
from jax.experimental import pallas as pl
from jax.experimental.pallas import tpu as pltpu
from jax.experimental.pallas import tpu_sc as plsc

NC = 2
NS = 16
L = 16
NW = NC * NS
S = 32
D = 128
NODES_PER_W = 320
NBLK = 3
BLK_ROWS = (128, 128, 64)
BLK_PAD = 128
N_PAD = NW * NODES_PER_W
N_TABLE = 10000
ROWS_PER_STAGER = 632
WINDOW = 16

_mesh = plsc.VectorSubcoreMesh(
    core_axis_name="c", subcore_axis_name="s", num_cores=NC)


N_OUT = 10000


@functools.partial(
    pl.kernel,
    out_type=(jax.ShapeDtypeStruct((N_OUT, D), jnp.float32),
              jax.ShapeDtypeStruct((BLK_PAD, D), jnp.float32)),
    mesh=_mesh,
    scratch_types=[
        pltpu.VMEM_SHARED((N_TABLE, D), jnp.float32),
        pltpu.VMEM((NBLK, S, BLK_PAD), jnp.int32),
        pltpu.VMEM((BLK_PAD, D), jnp.float32),
        pltpu.VMEM((BLK_PAD, D), jnp.float32),
        pltpu.SemaphoreType.DMA,
        pltpu.SemaphoreType.DMA,
        pltpu.SemaphoreType.DMA,
        pltpu.SemaphoreType.DMA,
        pltpu.SemaphoreType.DMA,
    ],
)
def _agg_kernel(idx_hbm, table_hbm, out_hbm, spill_hbm,
                table_sp, idx_v, acc_a, acc_b, gsem_a, gsem_b,
                fsem_a, fsem_b, tsem):
    sid = lax.axis_index("s")
    wid = sid * NC + lax.axis_index("c")
    out_base = wid * NODES_PER_W

    accs = (acc_a, acc_b)
    gsems = (gsem_a, gsem_b)
    fsems = (fsem_a, fsem_b)
    zeros = jnp.zeros((L,), jnp.float32)

    def zero_acc(nb):
        acc = accs[nb & 1]

        def zero_body(r, carry):
            for c in range(D // L):
                acc[r, pl.ds(c * L, L)] = zeros
            return carry

        lax.fori_loop(0, BLK_ROWS[nb], zero_body, 0, unroll=4)

    stage_base = pl.multiple_of(
        jnp.minimum(sid * ROWS_PER_STAGER, N_TABLE - ROWS_PER_STAGER), 8)
    stage = pltpu.make_async_copy(
        table_hbm.at[pl.ds(stage_base, ROWS_PER_STAGER)],
        table_sp.at[pl.ds(stage_base, ROWS_PER_STAGER)], tsem)
    stage.start()
    pltpu.sync_copy(idx_hbm.at[wid], idx_v)
    zero_acc(0)
    zero_acc(1)
    stage.wait()
    plsc.subcore_barrier()

    def gather_wait(b, n_rows):
        pltpu.make_async_copy(
            table_hbm.at[pl.ds(0, n_rows)],
            accs[b].at[pl.ds(0, n_rows)], gsems[b]).wait()

    def launch_block(nb, zero=False):
        b = nb & 1
        n_rows = BLK_ROWS[nb]
        acc = accs[b]
        dst = acc.at[pl.ds(0, n_rows)]
        if zero:
            zero_acc(nb)

        def pass_body(j, carry):
            pltpu.make_async_copy(
                table_sp.at[idx_v.at[nb, j, pl.ds(0, n_rows)]], dst,
                gsems[b]).start(add=True)

            @pl.when(j >= WINDOW)
            def _():
                gather_wait(b, n_rows)

            return carry

        lax.fori_loop(0, S, pass_body, 0)

    def finish_block(nb):
        b = nb & 1
        n_rows = BLK_ROWS[nb]
        acc = accs[b]
        for _ in range(WINDOW):
            gather_wait(b, n_rows)

        def scale_body(r, carry):
            for c in range(D // L):
                acc[r, pl.ds(c * L, L)] = acc[r, pl.ds(c * L, L)] * (1.0 / S)
            return carry

        lax.fori_loop(0, n_rows, scale_body, 0, unroll=4)

        @pl.when(wid < NW - 1)
        def _():
            pltpu.make_async_copy(
                acc.at[pl.ds(0, n_rows)],
                out_hbm.at[pl.ds(out_base + nb * BLK_PAD, n_rows)],
                fsems[b]).start()

        @pl.when(wid == NW - 1)
        def _():
            if nb == 0:
                pltpu.make_async_copy(
                    acc.at[pl.ds(0, N_OUT - (NW - 1) * NODES_PER_W)],
                    out_hbm.at[pl.ds((NW - 1) * NODES_PER_W,
                                     N_OUT - (NW - 1) * NODES_PER_W)],
                    fsems[b]).start()
            else:
                pltpu.make_async_copy(
                    acc.at[pl.ds(0, n_rows)],
                    spill_hbm.at[pl.ds(0, n_rows)], fsems[b]).start()

    def flush_wait(b, n_rows):
        pltpu.make_async_copy(
            accs[b].at[pl.ds(0, n_rows)],
            out_hbm.at[pl.ds(0, n_rows)], fsems[b]).wait()

    launch_block(0)
    launch_block(1)
    finish_block(0)
    @pl.when(wid < NW - 1)
    def _():
        flush_wait(0, BLK_ROWS[0])

    @pl.when(wid == NW - 1)
    def _():
        flush_wait(0, N_OUT - (NW - 1) * NODES_PER_W)

    launch_block(2, zero=True)
    finish_block(1)
    finish_block(2)
    flush_wait(1, BLK_ROWS[1])
    flush_wait(0, BLK_ROWS[2])


def kernel(neighbors, table):
    n, _ = neighbors.shape
    idx = neighbors.astype(jnp.int32)
    idx = jnp.pad(idx, ((0, N_PAD - n), (0, 0)))
    idx = idx.reshape(NW, NODES_PER_W, S)
    idx = jnp.pad(idx, ((0, 0), (0, NBLK * BLK_PAD - NODES_PER_W), (0, 0)))
    idx4 = idx.reshape(NW, NBLK, BLK_PAD, S).transpose(0, 1, 3, 2)
    out, _ = _agg_kernel(idx4, table)
    return out


# speedup vs baseline: 1.0038x; 1.0038x over previous
import functools

import jax
import jax.numpy as jnp
from jax import lax
from jax.experimental import pallas as pl
from jax.experimental.pallas import tpu as pltpu
from jax.experimental.pallas import tpu_sc as plsc

NC = 2
NS = 16
L = 16
NW = NC * NS
S = 32
D = 128
NODES_PER_W = 320
NBLK = 3
BLK_ROWS = (128, 128, 64)
BLK_PAD = 128
N_PAD = NW * NODES_PER_W
N_TABLE = 10000
ROWS_PER_STAGER = 632
WINDOW = 8

_mesh = plsc.VectorSubcoreMesh(
    core_axis_name="c", subcore_axis_name="s", num_cores=NC)


N_OUT = 10000


@functools.partial(
    pl.kernel,
    out_type=(jax.ShapeDtypeStruct((N_OUT, D), jnp.float32),
              jax.ShapeDtypeStruct((BLK_PAD, D), jnp.float32)),
    mesh=_mesh,
    scratch_types=[
        pltpu.VMEM_SHARED((N_TABLE, D), jnp.float32),
        pltpu.VMEM((NBLK, S, BLK_PAD), jnp.int32),
        pltpu.VMEM((BLK_PAD, D), jnp.float32),
        pltpu.VMEM((BLK_PAD, D), jnp.float32),
        pltpu.SemaphoreType.DMA,
        pltpu.SemaphoreType.DMA,
        pltpu.SemaphoreType.DMA,
        pltpu.SemaphoreType.DMA,
        pltpu.SemaphoreType.DMA,
    ],
)
def _agg_kernel(idx_hbm, table_hbm, out_hbm, spill_hbm,
                table_sp, idx_v, acc_a, acc_b, gsem_a, gsem_b,
                fsem_a, fsem_b, tsem):
    sid = lax.axis_index("s")
    wid = sid * NC + lax.axis_index("c")
    out_base = wid * NODES_PER_W

    accs = (acc_a, acc_b)
    gsems = (gsem_a, gsem_b)
    fsems = (fsem_a, fsem_b)
    zeros = jnp.zeros((L,), jnp.float32)

    def zero_acc(nb):
        acc = accs[nb & 1]

        def zero_body(r, carry):
            for c in range(D // L):
                acc[r, pl.ds(c * L, L)] = zeros
            return carry

        lax.fori_loop(0, BLK_ROWS[nb], zero_body, 0, unroll=4)

    stage_base = pl.multiple_of(
        jnp.minimum(sid * ROWS_PER_STAGER, N_TABLE - ROWS_PER_STAGER), 8)
    stage = pltpu.make_async_copy(
        table_hbm.at[pl.ds(stage_base, ROWS_PER_STAGER)],
        table_sp.at[pl.ds(stage_base, ROWS_PER_STAGER)], tsem)
    stage.start()
    pltpu.sync_copy(idx_hbm.at[wid], idx_v)
    zero_acc(0)
    zero_acc(1)
    stage.wait()
    plsc.subcore_barrier()

    def gather_wait(b, n_rows):
        pltpu.make_async_copy(
            table_hbm.at[pl.ds(0, n_rows)],
            accs[b].at[pl.ds(0, n_rows)], gsems[b]).wait()

    def launch_block(nb, zero=False):
        b = nb & 1
        n_rows = BLK_ROWS[nb]
        acc = accs[b]
        dst = acc.at[pl.ds(0, n_rows)]
        if zero:
            zero_acc(nb)

        def pass_body(j, carry):
            pltpu.make_async_copy(
                table_sp.at[idx_v.at[nb, j, pl.ds(0, n_rows)]], dst,
                gsems[b]).start(add=True)

            @pl.when(j >= WINDOW)
            def _():
                gather_wait(b, n_rows)

            return carry

        lax.fori_loop(0, S, pass_body, 0)

    def finish_block(nb):
        b = nb & 1
        n_rows = BLK_ROWS[nb]
        acc = accs[b]
        for _ in range(WINDOW):
            gather_wait(b, n_rows)

        def scale_body(r, carry):
            for c in range(D // L):
                acc[r, pl.ds(c * L, L)] = acc[r, pl.ds(c * L, L)] * (1.0 / S)
            return carry

        lax.fori_loop(0, n_rows, scale_body, 0, unroll=4)

        @pl.when(wid < NW - 1)
        def _():
            pltpu.make_async_copy(
                acc.at[pl.ds(0, n_rows)],
                out_hbm.at[pl.ds(out_base + nb * BLK_PAD, n_rows)],
                fsems[b]).start()

        @pl.when(wid == NW - 1)
        def _():
            if nb == 0:
                pltpu.make_async_copy(
                    acc.at[pl.ds(0, N_OUT - (NW - 1) * NODES_PER_W)],
                    out_hbm.at[pl.ds((NW - 1) * NODES_PER_W,
                                     N_OUT - (NW - 1) * NODES_PER_W)],
                    fsems[b]).start()
            else:
                pltpu.make_async_copy(
                    acc.at[pl.ds(0, n_rows)],
                    spill_hbm.at[pl.ds(0, n_rows)], fsems[b]).start()

    def flush_wait(b, n_rows):
        pltpu.make_async_copy(
            accs[b].at[pl.ds(0, n_rows)],
            out_hbm.at[pl.ds(0, n_rows)], fsems[b]).wait()

    launch_block(0)
    launch_block(1)
    finish_block(0)
    @pl.when(wid < NW - 1)
    def _():
        flush_wait(0, BLK_ROWS[0])

    @pl.when(wid == NW - 1)
    def _():
        flush_wait(0, N_OUT - (NW - 1) * NODES_PER_W)

    launch_block(2, zero=True)
    finish_block(1)
    finish_block(2)
    flush_wait(1, BLK_ROWS[1])
    flush_wait(0, BLK_ROWS[2])


def kernel(neighbors, table):
    n, _ = neighbors.shape
    idx = neighbors.astype(jnp.int32)
    idx = jnp.pad(idx, ((0, N_PAD - n), (0, 0)))
    idx = idx.reshape(NW, NODES_PER_W, S)
    idx = jnp.pad(idx, ((0, 0), (0, NBLK * BLK_PAD - NODES_PER_W), (0, 0)))
    idx4 = idx.reshape(NW, NBLK, BLK_PAD, S).transpose(0, 1, 3, 2)
    out, _ = _agg_kernel(idx4, table)
    return out
